# Initial kernel scaffold; baseline (speedup 1.0000x reference)
import jax
import jax.numpy as jnp
from jax.experimental import pallas as pl


def kernel(x):
    raise NotImplementedError("write your pallas kernel here")


# speedup vs baseline: 1.3347x; 1.3347x over previous
import functools

import jax
import jax.numpy as jnp
from jax import lax
from jax.experimental import pallas as pl
from jax.experimental.pallas import tpu as pltpu
from jax.experimental.pallas import tpu_sc as plsc

N = 16384
C = 26
CARD = 100
D = C * CARD
NW = 32
ROWS_PER_W = N // NW
G = 16
NGROUPS = ROWS_PER_W // G
TILE = G * D

_OUT_DTYPE = jax.dtypes.canonicalize_dtype(jnp.int64)

_mesh = plsc.VectorSubcoreMesh(core_axis_name="c", subcore_axis_name="s")


@functools.partial(
    pl.kernel,
    mesh=_mesh,
    compiler_params=pltpu.CompilerParams(needs_layout_passes=False),
    out_type=jax.ShapeDtypeStruct((N * D,), _OUT_DTYPE),
    scratch_types=[
        pltpu.VMEM((C, ROWS_PER_W), jnp.int32),
        pltpu.VMEM((TILE,), _OUT_DTYPE),
        pltpu.VMEM((TILE,), _OUT_DTYPE),
        pltpu.SemaphoreType.DMA,
        pltpu.SemaphoreType.DMA,
    ],
)
def _one_hot_sc(xt_hbm, out_hbm, xv, buf0, buf1, sem0, sem1):
    wid = lax.axis_index("s") * 2 + lax.axis_index("c")
    row_base = wid * ROWS_PER_W

    pltpu.sync_copy(xt_hbm.at[:, pl.ds(row_base * 1, ROWS_PER_W)], xv)

    lanes = lax.iota(jnp.int32, 16)
    row_off = lanes * D
    ones = jnp.full((16,), 1, _OUT_DTYPE)
    zeros = jnp.zeros((16,), _OUT_DTYPE)
    zvec = jnp.zeros((16,), _OUT_DTYPE)

    bufs = (buf0, buf1)
    sems = (sem0, sem1)

    def _zero(i, _):
        buf0[pl.ds(i * 16, 16)] = zvec
        buf1[pl.ds(i * 16, 16)] = zvec
        return _
    lax.fori_loop(0, TILE // 16, _zero, 0)

    def _scatter_group(buf, g, data):
        for c in range(C):
            vals = xv[c, pl.ds(g * G, G)]
            idx = row_off + (vals + (c * CARD))
            plsc.store_scatter(buf, [idx], data)

    def _dma_out(buf, sem, g):
        dst = out_hbm.at[pl.ds((row_base + g * G) * D, TILE)]
        pltpu.async_copy(buf, dst, sem)

    def _dma_wait(buf, sem):
        pltpu.make_async_copy(buf, out_hbm.at[pl.ds(0, TILE)], sem).wait()

    for b in range(2):
        _scatter_group(bufs[b], b, ones)
        _dma_out(bufs[b], sems[b], b)

    def _body(i, _):
        for b in range(2):
            g = i * 2 + 2 + b
            _dma_wait(bufs[b], sems[b])
            _scatter_group(bufs[b], g - 2, zeros)
            _scatter_group(bufs[b], g, ones)
            _dma_out(bufs[b], sems[b], g)
        return _
    lax.fori_loop(0, (NGROUPS - 2) // 2, _body, 0)

    for b in range(2):
        _dma_wait(bufs[b], sems[b])


def kernel(x):
    xt = jnp.asarray(x, jnp.int32).T.copy()
    flat = _one_hot_sc(xt)
    return flat.reshape(N, D)
